# Initial kernel scaffold; baseline (speedup 1.0000x reference)
import jax
import jax.numpy as jnp
from jax.experimental import pallas as pl


def kernel(x, edge_index, edge_weight, Wl, bl, Wr, br, We, att, bias_g, Wg, bg, Wo, bo):
    raise NotImplementedError("write your pallas kernel here")


# speedup vs baseline: 1.6927x; 1.6927x over previous
import functools

import jax
import jax.numpy as jnp
from jax.experimental import pallas as pl
from jax.experimental.pallas import tpu as pltpu

N = 10000
E = 160000
D = 128
C = 128
NP = 10240


def _mm_kernel(x_ref, w_ref, b_ref, o_ref):
    o_ref[...] = (
        jnp.dot(x_ref[...], w_ref[...], preferred_element_type=jnp.float32)
        + b_ref[...]
    )


def _mm(x, W, b, block=256):
    M, K = x.shape
    Cw = W.shape[1]
    return pl.pallas_call(
        _mm_kernel,
        grid=(M // block,),
        in_specs=[
            pl.BlockSpec((block, K), lambda i: (i, 0)),
            pl.BlockSpec((K, Cw), lambda i: (0, 0)),
            pl.BlockSpec((Cw,), lambda i: (0,)),
        ],
        out_specs=pl.BlockSpec((block, Cw), lambda i: (i, 0)),
        out_shape=jax.ShapeDtypeStruct((M, Cw), jnp.float32),
    )(x, W, b)


def kernel(x, edge_index, edge_weight, Wl, bl, Wr, br, We, att, bias_g, Wg, bg, Wo, bo):
    n = x.shape[0]
    src0, dst0 = edge_index[0], edge_index[1]
    loop = jnp.arange(n, dtype=src0.dtype)
    src = jnp.concatenate([src0, loop])
    dst = jnp.concatenate([dst0, loop])
    ei = jnp.stack([src, dst])

    xp = jnp.pad(x, ((0, NP - N), (0, 0)))
    XL = _mm(xp, Wl, bl)[:N]
    XR = _mm(xp, Wr, br)[:N]
    ewp = jnp.pad(edge_weight, ((0, 0), (0, 12)))
    Wep = jnp.pad(We, ((0, 12), (0, 0)))
    EW = _mm(ewp, Wep, jnp.zeros((C,), jnp.float32), block=320)
    EW = jnp.concatenate([EW, jnp.zeros((n, C), jnp.float32)], axis=0)

    mask = jnp.concatenate([src0 == dst0, jnp.zeros((n,), dtype=bool)])
    xj = XL[src]
    xi = XR[dst]
    m = jax.nn.leaky_relu(xi + xj + EW, 0.2)
    logits = (m * att[0][None, :]).sum(-1)
    ex = jnp.where(mask, 0.0, jnp.exp(logits))
    denom = jax.ops.segment_sum(ex, dst, num_segments=n)
    alpha = ex / denom[dst]
    h = jax.ops.segment_sum(xj * alpha[:, None], dst, num_segments=n) + bias_g
    h = jax.nn.relu(h)
    hg = _mm(jnp.pad(h, ((0, NP - N), (0, 0))), Wg, bg)[:N]
    h2 = jax.ops.segment_sum(hg[src] * alpha[:, None], dst, num_segments=n)
    h2 = jax.nn.relu(h2)
    out = _mm(jnp.pad(h2, ((0, NP - N), (0, 0))), jnp.pad(Wo, ((0, 0), (0, 126))), jnp.pad(bo, (0, 126)))[:N, :2]
    return (out, (ei, alpha[:, None]))


# speedup vs baseline: 2.7183x; 1.6059x over previous
import functools

import jax
import jax.numpy as jnp
from jax import lax
from jax.experimental import pallas as pl
from jax.experimental.pallas import tpu as pltpu
from jax.experimental.pallas import tpu_sc as plsc

N = 10000
E = 160000
D = 128
C = 128
NP = 10240
EP = 172032
NCHT = EP // 128
NW = 32
NCH = NCHT // NW
ROWS_PER_TILE = NP // 16


def _mm_kernel(x_ref, w_ref, b_ref, o_ref):
    o_ref[...] = (
        jnp.dot(x_ref[...], w_ref[...], preferred_element_type=jnp.float32)
        + b_ref[...]
    )


def _mm(x, W, b, block=256):
    M, K = x.shape
    Cw = W.shape[1]
    return pl.pallas_call(
        _mm_kernel,
        grid=(M // block,),
        in_specs=[
            pl.BlockSpec((block, K), lambda i: (i, 0)),
            pl.BlockSpec((K, Cw), lambda i: (0, 0)),
            pl.BlockSpec((Cw,), lambda i: (0,)),
        ],
        out_specs=pl.BlockSpec((block, Cw), lambda i: (i, 0)),
        out_shape=jax.ShapeDtypeStruct((M, Cw), jnp.float32),
    )(x, W, b)


def _seg_body(table, src2, dst2, scale2, out, sidx, didx, sbuf, rows, shared, semi, semg):
    cid = lax.axis_index("c")
    tid = lax.axis_index("s")
    wid = cid * 16 + tid

    def _zrow(r, _):
        for c in range(8):
            rows[r, pl.ds(c * 16, 16)] = jnp.zeros((16,), jnp.float32)
        return 0
    lax.fori_loop(0, 128, _zrow, 0)
    for j in range(ROWS_PER_TILE // 128):
        pltpu.sync_copy(rows, shared.at[pl.ds(tid * ROWS_PER_TILE + j * 128, 128)])
    plsc.subcore_barrier()

    def _chunk(c, _):
        row = wid * NCH + c
        cp1 = pltpu.async_copy(src2.at[row], sidx, semi)
        cp2 = pltpu.async_copy(dst2.at[row], didx, semi)
        cp3 = pltpu.async_copy(scale2.at[row], sbuf, semi)
        cp1.wait()
        cp2.wait()
        cp3.wait()
        pltpu.async_copy(table.at[sidx], rows, semg).wait()

        def _grp(g, _):
            s16 = sbuf[pl.ds(g * 16, 16)]
            for i in range(16):
                e = g * 16 + i
                ss = jnp.broadcast_to(s16[i:i + 1], (16,))
                for cc in range(8):
                    rows[e, pl.ds(cc * 16, 16)] = rows[e, pl.ds(cc * 16, 16)] * ss
            return 0
        lax.fori_loop(0, 8, _grp, 0)
        pltpu.sync_copy(rows, shared.at[didx], add=True)
        return 0
    lax.fori_loop(0, NCH, _chunk, 0)

    plsc.subcore_barrier()
    pltpu.sync_copy(
        shared.at[pl.ds(tid * ROWS_PER_TILE, ROWS_PER_TILE)],
        out.at[cid].at[pl.ds(tid * ROWS_PER_TILE, ROWS_PER_TILE)],
    )


@functools.partial(jax.jit, static_argnames=())
def _seg_gather_scale_scatter(table, src2, dst2, scale2):
    mesh = plsc.VectorSubcoreMesh(core_axis_name="c", subcore_axis_name="s")
    f = pl.kernel(
        _seg_body,
        out_type=jax.ShapeDtypeStruct((2, NP, 128), jnp.float32),
        mesh=mesh,
        scratch_types=[
            pltpu.VMEM((128,), jnp.int32),
            pltpu.VMEM((128,), jnp.int32),
            pltpu.VMEM((128,), jnp.float32),
            pltpu.VMEM((128, 128), jnp.float32),
            pltpu.VMEM_SHARED((NP, 128), jnp.float32),
            pltpu.SemaphoreType.DMA,
            pltpu.SemaphoreType.DMA,
        ],
    )
    return f(table, src2, dst2, scale2)


def kernel(x, edge_index, edge_weight, Wl, bl, Wr, br, We, att, bias_g, Wg, bg, Wo, bo):
    n = x.shape[0]
    src0, dst0 = edge_index[0], edge_index[1]
    loop = jnp.arange(n, dtype=src0.dtype)
    src = jnp.concatenate([src0, loop])
    dst = jnp.concatenate([dst0, loop])
    ei = jnp.stack([src, dst])
    pad = EP - (E + N)
    src2 = jnp.concatenate([src, jnp.zeros((pad,), jnp.int32)]).reshape(NCHT, 128)
    dst2 = jnp.concatenate([dst, jnp.zeros((pad,), jnp.int32)]).reshape(NCHT, 128)

    xp = jnp.pad(x, ((0, NP - N), (0, 0)))
    XL = _mm(xp, Wl, bl)[:N]
    XR = _mm(xp, Wr, br)[:N]
    ewp = jnp.pad(edge_weight, ((0, 0), (0, 12)))
    Wep = jnp.pad(We, ((0, 12), (0, 0)))
    EW = _mm(ewp, Wep, jnp.zeros((C,), jnp.float32), block=320)
    EW = jnp.concatenate([EW, jnp.zeros((n, C), jnp.float32)], axis=0)

    mask = jnp.concatenate([src0 == dst0, jnp.zeros((n,), dtype=bool)])
    xj = XL[src]
    xi = XR[dst]
    m = jax.nn.leaky_relu(xi + xj + EW, 0.2)
    logits = (m * att[0][None, :]).sum(-1)
    ex = jnp.where(mask, 0.0, jnp.exp(logits))
    denom = jax.ops.segment_sum(ex, dst, num_segments=n)
    alpha = ex / denom[dst]
    al2 = jnp.concatenate([alpha, jnp.zeros((pad,), jnp.float32)]).reshape(NCHT, 128)

    hp = _seg_gather_scale_scatter(XL, src2, dst2, al2)
    h = (hp[0] + hp[1])[:N] + bias_g
    h = jax.nn.relu(h)
    hg = _mm(jnp.pad(h, ((0, NP - N), (0, 0))), Wg, bg)[:N]
    h2p = _seg_gather_scale_scatter(hg, src2, dst2, al2)
    h2 = jax.nn.relu((h2p[0] + h2p[1])[:N])
    out = _mm(jnp.pad(h2, ((0, NP - N), (0, 0))), jnp.pad(Wo, ((0, 0), (0, 126))), jnp.pad(bo, (0, 126)))[:N, :2]
    return (out, (ei, alpha[:, None]))


# speedup vs baseline: 4.3846x; 1.6130x over previous
import functools

import jax
import jax.numpy as jnp
from jax import lax
from jax.experimental import pallas as pl
from jax.experimental.pallas import tpu as pltpu
from jax.experimental.pallas import tpu_sc as plsc

N = 10000
E = 160000
D = 128
C = 128
NP = 10240
EPR = 163840
NCHA = EPR // 128
NW = 32
NCH = NCHA // NW


def _mm_kernel(x_ref, w_ref, b_ref, o_ref):
    o_ref[...] = (
        jnp.dot(x_ref[...], w_ref[...], preferred_element_type=jnp.float32)
        + b_ref[...]
    )


def _mm(x, W, b, block=256):
    M, K = x.shape
    Cw = W.shape[1]
    return pl.pallas_call(
        _mm_kernel,
        grid=(M // block,),
        in_specs=[
            pl.BlockSpec((block, K), lambda i: (i, 0)),
            pl.BlockSpec((K, Cw), lambda i: (0, 0)),
            pl.BlockSpec((Cw,), lambda i: (0,)),
        ],
        out_specs=pl.BlockSpec((block, Cw), lambda i: (i, 0)),
        out_shape=jax.ShapeDtypeStruct((M, Cw), jnp.float32),
    )(x, W, b)


def _hstage_kernel(hp_ref, dp_ref, xl_ref, xr_ref, att_ref, bias_ref, wg_ref,
                   bgr_ref, hg_ref, invd_ref, aloop_ref):
    xl = xl_ref[...]
    zl = jax.nn.leaky_relu(xl + xr_ref[...], 0.2)
    exl = jnp.exp(jnp.sum(zl * att_ref[...], axis=1))
    denom = dp_ref[0, :, 0] + dp_ref[1, :, 0] + exl
    invd = 1.0 / denom
    hs = (hp_ref[0] + hp_ref[1] + exl[:, None] * xl) * invd[:, None]
    h = jax.nn.relu(hs + bias_ref[...])
    hg_ref[...] = jnp.dot(h, wg_ref[...], preferred_element_type=jnp.float32)
    invd_ref[...] = jnp.broadcast_to(invd[:, None], (invd.shape[0], 16))
    aloop_ref[...] = exl * invd


def _hstage(hp, denp, XL, XR, att1, bias_g, Wg, bg, block=256):
    return pl.pallas_call(
        _hstage_kernel,
        grid=(NP // block,),
        in_specs=[
            pl.BlockSpec((2, block, D), lambda i: (0, i, 0)),
            pl.BlockSpec((2, block, D), lambda i: (0, i, 0)),
            pl.BlockSpec((block, D), lambda i: (i, 0)),
            pl.BlockSpec((block, D), lambda i: (i, 0)),
            pl.BlockSpec((1, D), lambda i: (0, 0)),
            pl.BlockSpec((D,), lambda i: (0,)),
            pl.BlockSpec((D, D), lambda i: (0, 0)),
            pl.BlockSpec((D,), lambda i: (0,)),
        ],
        out_specs=[
            pl.BlockSpec((block, D), lambda i: (i, 0)),
            pl.BlockSpec((block, 16), lambda i: (i, 0)),
            pl.BlockSpec((block,), lambda i: (i,)),
        ],
        out_shape=[
            jax.ShapeDtypeStruct((NP, D), jnp.float32),
            jax.ShapeDtypeStruct((NP, 16), jnp.float32),
            jax.ShapeDtypeStruct((NP,), jnp.float32),
        ],
    )(hp, denp, XL, XR, att1, bias_g, Wg, bg)


def _ostage_kernel(qp_ref, iv_ref, hg_ref, aloop_ref, bg2_ref, wo_ref, bo_ref, o_ref):
    q = (qp_ref[0] + qp_ref[1]) * iv_ref[:, 0:1] + aloop_ref[...][:, None] * hg_ref[...]
    h2 = jax.nn.relu(q + bg2_ref[...])
    o_ref[...] = jnp.dot(h2, wo_ref[...], preferred_element_type=jnp.float32) + bo_ref[...]


def _ostage(qp, invd16, hg, aloop, bg2, Wop, bop, block=256):
    return pl.pallas_call(
        _ostage_kernel,
        grid=(NP // block,),
        in_specs=[
            pl.BlockSpec((2, block, D), lambda i: (0, i, 0)),
            pl.BlockSpec((block, 16), lambda i: (i, 0)),
            pl.BlockSpec((block, D), lambda i: (i, 0)),
            pl.BlockSpec((block,), lambda i: (i,)),
            pl.BlockSpec((D,), lambda i: (0,)),
            pl.BlockSpec((D, D), lambda i: (0, 0)),
            pl.BlockSpec((D,), lambda i: (0,)),
        ],
        out_specs=pl.BlockSpec((block, D), lambda i: (i, 0)),
        out_shape=jax.ShapeDtypeStruct((NP, D), jnp.float32),
    )(qp, invd16, hg, aloop, bg2, Wop, bop)


def _passA_body(xl, xr, src2, dst2, ew4, web, hp, ex2,
                sidx, didx, ewb, wewb, xlr, xrr, exb, shared, semi, semg):
    cid = lax.axis_index("c")
    tid = lax.axis_index("s")
    wid = cid * 16 + tid

    pltpu.sync_copy(web, wewb)

    def _zrow(r, _):
        for c in range(8):
            xlr[r, pl.ds(c * 16, 16)] = jnp.zeros((16,), jnp.float32)
        return 0
    lax.fori_loop(0, 128, _zrow, 0)

    @pl.when(tid < 15)
    def _():
        for j in range(5):
            pltpu.sync_copy(xlr, shared.at[pl.ds(tid * 640 + j * 128, 128)])

    @pl.when(tid == 15)
    def _():
        for j in range(3):
            pltpu.sync_copy(xlr, shared.at[pl.ds(9600 + j * 128, 128)])
        pltpu.sync_copy(xlr.at[pl.ds(0, 16)], shared.at[pl.ds(9984, 16)])
    plsc.subcore_barrier()

    def _chunk(c, _):
        row = wid * NCH + c
        cp1 = pltpu.async_copy(src2.at[row], sidx, semi)
        cp2 = pltpu.async_copy(dst2.at[row], didx, semi)
        cp3 = pltpu.async_copy(ew4.at[row], ewb.at[pl.ds(0, 512)], semi)
        cp1.wait(); cp2.wait(); cp3.wait()
        g1 = pltpu.async_copy(xl.at[sidx], xlr, semg)
        g2 = pltpu.async_copy(xr.at[didx], xrr, semg)
        g1.wait(); g2.wait()

        def _grp(g, _):
            we_regs = [[wewb[k, pl.ds(cc * 16, 16)] for cc in range(8)] for k in range(4)]
            attv = [wewb[4, pl.ds(cc * 16, 16)] for cc in range(8)]
            iota = lax.iota(jnp.int32, 16)

            def _xorp(a, k):
                return a.at[iota ^ k].get(mode="promise_in_bounds")

            def _comb(a, b, k):
                pa = a + _xorp(a, k)
                pb = b + _xorp(b, k)
                return jnp.where((iota & k) == 0, pa, pb)

            s16 = sidx[pl.ds(g * 16, 16)]
            d16 = didx[pl.ds(g * 16, 16)]
            stack = []
            for i in range(16):
                e = g * 16 + i
                wv = ewb[pl.ds(e * 4, 16)]
                acc = jnp.zeros((16,), jnp.float32)
                w = [jnp.broadcast_to(wv[k:k + 1], (16,)) for k in range(4)]
                for cc in range(8):
                    z = xlr[e, pl.ds(cc * 16, 16)] + xrr[e, pl.ds(cc * 16, 16)]
                    for k in range(4):
                        z = z + w[k] * we_regs[k][cc]
                    z = jnp.maximum(z, 0.2 * z)
                    acc = acc + z * attv[cc]
                node, lvl = acc, 0
                while stack and stack[-1][0] == lvl:
                    _, prev = stack.pop()
                    node = _comb(prev, node, 1 << lvl)
                    lvl += 1
                stack.append((lvl, node))
            lg16 = stack[0][1]
            ex16 = jnp.where(s16 == d16, 0.0, jnp.exp(lg16))
            exb[pl.ds(g * 16, 16)] = ex16
            return 0
        lax.fori_loop(0, 8, _grp, 0)

        def _scale(g, _):
            ex16 = exb[pl.ds(g * 16, 16)]
            iota = lax.iota(jnp.int32, 16)
            for i in range(16):
                e = g * 16 + i
                ss = jnp.broadcast_to(ex16[i:i + 1], (16,))
                for cc in range(8):
                    xlr[e, pl.ds(cc * 16, 16)] = xlr[e, pl.ds(cc * 16, 16)] * ss
            return 0
        lax.fori_loop(0, 8, _scale, 0)

        pltpu.sync_copy(exb, ex2.at[row])
        pltpu.sync_copy(xlr, shared.at[didx], add=True)
        return 0
    lax.fori_loop(0, NCH, _chunk, 0)

    plsc.subcore_barrier()

    @pl.when(tid < 15)
    def _():
        pltpu.sync_copy(shared.at[pl.ds(tid * 640, 640)],
                        hp.at[cid].at[pl.ds(tid * 640, 640)])

    @pl.when(tid == 15)
    def _():
        pltpu.sync_copy(shared.at[pl.ds(9600, 400)],
                        hp.at[cid].at[pl.ds(9600, 400)])


def _passA(XL, XR, src2, dst2, ew4, web):
    mesh = plsc.VectorSubcoreMesh(core_axis_name="c", subcore_axis_name="s")
    f = pl.kernel(
        _passA_body,
        out_type=[
            jax.ShapeDtypeStruct((2, NP, D), jnp.float32),
            jax.ShapeDtypeStruct((NCHA, 128), jnp.float32),
        ],
        mesh=mesh,
        scratch_types=[
            pltpu.VMEM((128,), jnp.int32),
            pltpu.VMEM((128,), jnp.int32),
            pltpu.VMEM((528,), jnp.float32),
            pltpu.VMEM((5, 128), jnp.float32),
            pltpu.VMEM((128, D), jnp.float32),
            pltpu.VMEM((128, D), jnp.float32),
            pltpu.VMEM((128,), jnp.float32),
            pltpu.VMEM_SHARED((N, D), jnp.float32),
            pltpu.SemaphoreType.DMA,
            pltpu.SemaphoreType.DMA,
        ],
    )
    return f(XL, XR, src2, dst2, ew4, web)


def _passDen_body(dst2, ex2, denp, didx, exb, exrow, dennacc, semi):
    cid = lax.axis_index("c")
    tid = lax.axis_index("s")
    wid = cid * 16 + tid

    def _zd(r, _):
        for c in range(8):
            exrow[r, pl.ds(c * 16, 16)] = jnp.zeros((16,), jnp.float32)
        return 0
    lax.fori_loop(0, 128, _zd, 0)

    @pl.when(tid < 15)
    def _():
        for j in range(5):
            pltpu.sync_copy(exrow, dennacc.at[pl.ds(tid * 640 + j * 128, 128)])

    @pl.when(tid == 15)
    def _():
        for j in range(3):
            pltpu.sync_copy(exrow, dennacc.at[pl.ds(9600 + j * 128, 128)])
        pltpu.sync_copy(exrow.at[pl.ds(0, 16)], dennacc.at[pl.ds(9984, 16)])
    plsc.subcore_barrier()

    def _chunk(c, _):
        row = wid * NCH + c
        cp1 = pltpu.async_copy(dst2.at[row], didx, semi)
        cp2 = pltpu.async_copy(ex2.at[row], exb, semi)
        cp1.wait(); cp2.wait()

        def _grp(g, _):
            ex16 = exb[pl.ds(g * 16, 16)]
            iota = lax.iota(jnp.int32, 16)
            for i in range(16):
                e = g * 16 + i
                ss = jnp.broadcast_to(ex16[i:i + 1], (16,))
                exrow[e, pl.ds(0, 16)] = jnp.where(iota == 0, ss, 0.0)
            return 0
        lax.fori_loop(0, 8, _grp, 0)
        pltpu.sync_copy(exrow, dennacc.at[didx], add=True)
        return 0
    lax.fori_loop(0, NCH, _chunk, 0)

    plsc.subcore_barrier()

    @pl.when(tid < 15)
    def _():
        pltpu.sync_copy(dennacc.at[pl.ds(tid * 640, 640)],
                        denp.at[cid].at[pl.ds(tid * 640, 640)])

    @pl.when(tid == 15)
    def _():
        pltpu.sync_copy(dennacc.at[pl.ds(9600, 400)],
                        denp.at[cid].at[pl.ds(9600, 400)])


def _passDen(dst2, ex2):
    mesh = plsc.VectorSubcoreMesh(core_axis_name="c", subcore_axis_name="s")
    f = pl.kernel(
        _passDen_body,
        out_type=jax.ShapeDtypeStruct((2, NP, D), jnp.float32),
        mesh=mesh,
        scratch_types=[
            pltpu.VMEM((128,), jnp.int32),
            pltpu.VMEM((128,), jnp.float32),
            pltpu.VMEM((128, D), jnp.float32),
            pltpu.VMEM_SHARED((N, D), jnp.float32),
            pltpu.SemaphoreType.DMA,
        ],
    )
    return f(dst2, ex2)


def _passC_body(hg, src2, dst2, ex2, h2p,
                sidx, didx, exb, rows, shared, semi, semg):
    cid = lax.axis_index("c")
    tid = lax.axis_index("s")
    wid = cid * 16 + tid

    def _zrow(r, _):
        for c in range(8):
            rows[r, pl.ds(c * 16, 16)] = jnp.zeros((16,), jnp.float32)
        return 0
    lax.fori_loop(0, 128, _zrow, 0)

    @pl.when(tid < 15)
    def _():
        for j in range(5):
            pltpu.sync_copy(rows, shared.at[pl.ds(tid * 640 + j * 128, 128)])

    @pl.when(tid == 15)
    def _():
        for j in range(3):
            pltpu.sync_copy(rows, shared.at[pl.ds(9600 + j * 128, 128)])
        pltpu.sync_copy(rows.at[pl.ds(0, 16)], shared.at[pl.ds(9984, 16)])
    plsc.subcore_barrier()

    def _chunk(c, _):
        row = wid * NCH + c
        cp1 = pltpu.async_copy(src2.at[row], sidx, semi)
        cp2 = pltpu.async_copy(dst2.at[row], didx, semi)
        cp3 = pltpu.async_copy(ex2.at[row], exb, semi)
        cp1.wait(); cp2.wait(); cp3.wait()
        pltpu.async_copy(hg.at[sidx], rows, semg).wait()

        def _grp(g, _):
            ex16 = exb[pl.ds(g * 16, 16)]
            for i in range(16):
                e = g * 16 + i
                ss = jnp.broadcast_to(ex16[i:i + 1], (16,))
                for cc in range(8):
                    rows[e, pl.ds(cc * 16, 16)] = rows[e, pl.ds(cc * 16, 16)] * ss
            return 0
        lax.fori_loop(0, 8, _grp, 0)

        pltpu.sync_copy(rows, shared.at[didx], add=True)
        return 0
    lax.fori_loop(0, NCH, _chunk, 0)

    plsc.subcore_barrier()

    @pl.when(tid < 15)
    def _():
        pltpu.sync_copy(shared.at[pl.ds(tid * 640, 640)],
                        h2p.at[cid].at[pl.ds(tid * 640, 640)])

    @pl.when(tid == 15)
    def _():
        pltpu.sync_copy(shared.at[pl.ds(9600, 400)],
                        h2p.at[cid].at[pl.ds(9600, 400)])


def _passC(hg, src2, dst2, ex2):
    mesh = plsc.VectorSubcoreMesh(core_axis_name="c", subcore_axis_name="s")
    f = pl.kernel(
        _passC_body,
        out_type=jax.ShapeDtypeStruct((2, NP, D), jnp.float32),
        mesh=mesh,
        scratch_types=[
            pltpu.VMEM((128,), jnp.int32),
            pltpu.VMEM((128,), jnp.int32),
            pltpu.VMEM((128,), jnp.float32),
            pltpu.VMEM((128, D), jnp.float32),
            pltpu.VMEM_SHARED((N, D), jnp.float32),
            pltpu.SemaphoreType.DMA,
            pltpu.SemaphoreType.DMA,
        ],
    )
    return f(hg, src2, dst2, ex2)


def kernel(x, edge_index, edge_weight, Wl, bl, Wr, br, We, att, bias_g, Wg, bg, Wo, bo):
    n = x.shape[0]
    src0, dst0 = edge_index[0], edge_index[1]
    loop = jnp.arange(n, dtype=src0.dtype)
    ei = jnp.stack([jnp.concatenate([src0, loop]), jnp.concatenate([dst0, loop])])

    pad = EPR - E
    src2 = jnp.concatenate([src0, jnp.zeros((pad,), jnp.int32)]).reshape(NCHA, 128)
    dst2 = jnp.concatenate([dst0, jnp.zeros((pad,), jnp.int32)]).reshape(NCHA, 128)
    ew4 = jnp.concatenate(
        [edge_weight, jnp.zeros((pad, 4), jnp.float32)]).reshape(NCHA, 512)
    web = jnp.concatenate([We, att], axis=0)

    xp = jnp.pad(x, ((0, NP - N), (0, 0)))
    XL = _mm(xp, Wl, bl)
    XR = _mm(xp, Wr, br)

    hp, ex2 = _passA(XL, XR, src2, dst2, ew4, web)
    denp = _passDen(dst2, ex2)
    hg, invd16, aloop = _hstage(hp, denp, XL, XR, att, bias_g, Wg, bg)
    h2p = _passC(hg, src2, dst2, ex2)
    out = _ostage(h2p, invd16, hg, aloop, bg, jnp.pad(Wo, ((0, 0), (0, 126))),
                  jnp.pad(bo, (0, 126)))[:N, :2]

    al_edges = ex2.reshape(EPR)[:E] * invd16[:, 0][dst0]
    alpha = jnp.concatenate([al_edges, aloop[:N]])
    return (out, (ei, alpha[:, None]))
